# baseline (speedup 1.0000x reference)
import jax
import jax.numpy as jnp
from jax.experimental import pallas as pl
from jax.experimental.pallas import tpu as pltpu

_ALPHA = 0.007
_BETA = 0.016
_OMEGA_O = 0.57
_OMEGA_T = 0.41
_NO_VALUE = -999.0
_EDGE_W = (1.0, 0.98, 0.97, 0.96, 0.95)

_B, _T, _C, _H, _W = 4, 12, 1, 512, 512
_NUMEL = _B * _C * _H * _W
_K90_LO = 943717
_K13 = 349525

_IMIN = -2147483648
_IMAX = 2147483647


def _key_of(v):
    u = jax.lax.bitcast_convert_type(v, jnp.int32)
    return jnp.where(u >= 0, u, _IMIN - u)


def _val_of(k):
    u = jnp.where(k >= 0, k, _IMIN - k)
    return jax.lax.bitcast_convert_type(u.astype(jnp.int32), jnp.float32)


def _select_rank(yt, k90, k13):
    ninf = jnp.float32(-jnp.inf)
    pinf = jnp.float32(jnp.inf)
    imin = jnp.int32(_IMIN)
    n = jnp.int32(_NUMEL)
    klo0 = _key_of(ninf)
    khi0 = _key_of(pinf)

    def init():
        return (klo0, khi0, jnp.int32(0), n,
                jnp.float32(-6.5), jnp.float32(6.5))

    def done(st, k):
        lo, hi, clo, chi, _, _ = st
        return (chi == k + 1) | (clo == k) | ((hi - lo) == 1)

    def midpoint(st):
        lo, hi = st[0], st[1]
        gap = hi - lo
        half = jax.lax.shift_right_logical(gap, 1)
        return jax.lax.bitwise_xor(
            jax.lax.bitwise_xor(lo, imin) + half, imin)

    def secant(st, k, r, guess):
        lo, hi, clo, chi, vlo, vhi = st
        frac = (jnp.float32(k + 1) - clo.astype(jnp.float32)) / (
            chi.astype(jnp.float32) - clo.astype(jnp.float32))
        tv = jnp.where(r == 0, guess, vlo + (vhi - vlo) * frac)
        return jnp.minimum(jnp.maximum(_key_of(tv), lo + 1), hi - 1)

    def update(st, k, t, cnt):
        lo, hi, clo, chi, vlo, vhi = st
        act = jnp.logical_not(done(st, k))
        up = act & (cnt >= k + 1)
        dn = act & (cnt < k + 1)
        tval = _val_of(t)
        return (jnp.where(dn, t, lo), jnp.where(up, t, hi),
                jnp.where(dn, cnt, clo), jnp.where(up, cnt, chi),
                jnp.where(dn, tval, vlo), jnp.where(up, tval, vhi))

    def round_pair(sts, t90, t13):
        st90, st13 = sts
        c90 = jnp.sum((yt <= _val_of(t90)).astype(jnp.int32))
        c13 = jnp.sum((yt <= _val_of(t13)).astype(jnp.int32))
        return (update(st90, k90, t90, c90), update(st13, k13, t13, c13))

    def secant_body(r, sts):
        t90 = secant(sts[0], k90, r, jnp.float32(1.2815516))
        t13 = secant(sts[1], k13, r, jnp.float32(-0.4307273))
        return round_pair(sts, t90, t13)

    def mid_body(r, sts):
        return round_pair(sts, midpoint(sts[0]), midpoint(sts[1]))

    sts = (init(), init())
    sts = jax.lax.fori_loop(0, 7, secant_body, sts)
    both = lambda s: done(s[0], k90) & done(s[1], k13)
    sts = jax.lax.cond(both(sts), lambda s: s,
                       lambda s: jax.lax.fori_loop(0, 8, mid_body, s), sts)
    sts = jax.lax.cond(both(sts), lambda s: s,
                       lambda s: jax.lax.fori_loop(0, 32, mid_body, s), sts)
    st90, st13 = sts

    def extract(st, k, with_nxt):
        lo, hi, clo, chi, _, _ = st
        m_gt = jnp.min(jnp.where(yt > _val_of(lo), yt, pinf))
        m_le = jnp.max(jnp.where(yt <= _val_of(hi), yt, ninf))
        ans = jnp.where((hi - lo) == 1, _val_of(hi),
                        jnp.where(clo == k, m_gt, m_le))
        if not with_nxt:
            return ans
        m_nxt = jnp.min(jnp.where(yt > _val_of(hi), yt, pinf))
        need_extra = (clo == k) & (chi != k + 1) & ((hi - lo) != 1)

        def extra(_):
            c = jnp.sum((yt <= ans).astype(jnp.int32))
            nx = jnp.min(jnp.where(yt > ans, yt, pinf))
            return c, nx

        c_a, nxt = jax.lax.cond(need_extra, extra,
                                lambda _: (chi, m_nxt), 0)
        return ans, c_a, nxt

    return extract(st90, k90, True), extract(st13, k13, False)


def _edge_weight(idx):
    e = jnp.zeros_like(idx, dtype=jnp.float32)
    for i, w in enumerate(_EDGE_W):
        e = e + jnp.where(idx == i, w, 0.0) + jnp.where(idx == (_H - 1 - i), w, 0.0)
    return e


def _min_weight(m):
    e = jnp.zeros_like(m, dtype=jnp.float32)
    for i, w in enumerate(_EDGE_W):
        e = e + jnp.where(m == i, w, 0.0)
    return e


def _loss_kernel(yp_ref, yt_ref, out_ref, bm_ref):
    yt = yt_ref[...]
    yp = yp_ref[...]

    @pl.when(pl.program_id(0) == 0)
    def _():
        h = jax.lax.broadcasted_iota(jnp.int32, (_H, _W), 0)
        w = jax.lax.broadcasted_iota(jnp.int32, (_H, _W), 1)
        diag = (h == w) | (h + w == _H - 1)
        bm_ref[...] = _edge_weight(h) + _edge_weight(w) + jnp.where(
            diag, _min_weight(jnp.minimum(h, _H - 1 - h)), 0.0)

    bmask = bm_ref[...][None, None, None, :, :]

    (va, c_a, nxt), q13 = _select_rank(yt, _K90_LO, _K13)
    vb = jnp.where(c_a >= _K90_LO + 2, va, nxt)
    q90 = va + (vb - va) * jnp.float32(0.5)

    diff = jnp.abs(yt - yp)
    no_value = yt == _NO_VALUE
    outlier = yt > q90
    normal = jnp.logical_not(no_value | outlier)
    over = yp >= yt
    torr = (yt >= q13) & normal
    wsq = (_ALPHA * jnp.exp(_BETA * yt)) * ((yt - yp) * (yt - yp))

    zero = jnp.float32(0.0)
    om_o = jnp.float32(_OMEGA_O)
    om_o1 = jnp.float32(1.0 - _OMEGA_O)
    om_t = jnp.float32(_OMEGA_T)
    om_t1 = jnp.float32(1.0 - _OMEGA_T)
    coef_d = (jnp.where(no_value, om_o, zero)
              + jnp.where(outlier, om_o1, zero)
              + om_o1 * bmask
              + jnp.where(normal, jnp.where(over, om_o1, om_o), zero))
    coef_w = jnp.where(torr, jnp.where(over, om_t1, om_t), zero)
    loss_sum = jnp.sum(coef_d * diff + coef_w * wsq)
    s_abs = jnp.sum(jnp.where(no_value, zero, diff))
    s_cnt = jnp.sum(jnp.where(no_value, zero, jnp.float32(1.0)))
    loss_t = loss_sum * jnp.float32(1.0 / _NUMEL)

    r = jax.lax.broadcasted_iota(jnp.int32, (1, 8, 128), 1)
    c = jax.lax.broadcasted_iota(jnp.int32, (1, 8, 128), 2)
    first = r == 0
    tile = (jnp.where(first & (c == 0), loss_t, zero)
            + jnp.where(first & (c == 1), s_abs, zero)
            + jnp.where(first & (c == 2), s_cnt, zero))
    out_ref[...] = tile


@jax.jit
def kernel(y_pred, y_true):
    block = (_B, 1, _C, _H, _W)
    partials = pl.pallas_call(
        _loss_kernel,
        grid=(_T,),
        in_specs=[
            pl.BlockSpec(block, lambda t: (0, t, 0, 0, 0)),
            pl.BlockSpec(block, lambda t: (0, t, 0, 0, 0)),
        ],
        out_specs=pl.BlockSpec((1, 8, 128), lambda t: (t, 0, 0)),
        out_shape=jax.ShapeDtypeStruct((_T, 8, 128), jnp.float32),
        scratch_shapes=[pltpu.VMEM((_H, _W), jnp.float32)],
    )(y_pred, y_true)

    losses = partials[:, 0, 0]
    s_abs = partials[:, 0, 1]
    s_cnt = partials[:, 0, 2]
    seasons = ((0, 1, 11), (2, 3, 4), (5, 6, 7), (8, 9, 10))
    seasonal = jnp.float32(0.0)
    for idx in seasons:
        ii = jnp.asarray(idx)
        seasonal = seasonal + jnp.sum(s_abs[ii]) / jnp.sum(s_cnt[ii])
    return jnp.mean(losses) + seasonal
